# baseline (speedup 1.0000x reference)
import functools

import jax
import jax.numpy as jnp
from jax import lax
from jax.experimental import pallas as pl
from jax.experimental.pallas import tpu as pltpu
from jax.experimental.pallas import tpu_sc as plsc

EPS = 1e-5
D = 256
L = 512
CH = 128
NCH = L // CH
RPW = 8
NLANE = 16


def _stats_body(tok_ref, pos_ref, row_ref, negmu_ref, rstd_ref):
    tok = tok_ref[...]
    pos = pos_ref[...]
    row = row_ref[...]
    muT = jnp.mean(tok, axis=1)
    muP = jnp.mean(pos, axis=1)
    muR = jnp.mean(row, axis=1)
    St = jnp.sum(tok * tok, axis=1)
    Sp = jnp.sum(pos * pos, axis=1)
    Sr = jnp.sum(row * row, axis=1)
    cdims = (((1,), (1,)), ((), ()))
    PT = lax.dot_general(tok, pos, cdims, preferred_element_type=jnp.float32)
    PR = lax.dot_general(row, pos, cdims, preferred_element_type=jnp.float32)
    RT = lax.dot_general(row, tok, cdims, preferred_element_type=jnp.float32)
    mu = muR[:, None, None] + muT[None, :, None] + muP[None, None, :]
    S = (Sr[:, None, None] + St[None, :, None] + Sp[None, None, :]
         + 2.0 * (RT[:, :, None] + PT[None, :, :] + PR[:, None, :]))
    var = S * (1.0 / D) - mu * mu
    negmu_ref[0] = -mu
    rstd_ref[0] = lax.rsqrt(var + EPS)


def _stats_tables(token_table, pos_table, row_table):
    V = token_table.shape[0]
    N = row_table.shape[0]
    tok8 = jnp.zeros((8, D), jnp.float32).at[:V].set(token_table)
    negmu, rstd = pl.pallas_call(
        _stats_body,
        grid=(NCH,),
        in_specs=[
            pl.BlockSpec((8, D), lambda c: (0, 0)),
            pl.BlockSpec((CH, D), lambda c: (c, 0)),
            pl.BlockSpec((N, D), lambda c: (0, 0)),
        ],
        out_specs=[
            pl.BlockSpec((1, N, 8, CH), lambda c: (c, 0, 0, 0)),
            pl.BlockSpec((1, N, 8, CH), lambda c: (c, 0, 0, 0)),
        ],
        out_shape=[
            jax.ShapeDtypeStruct((NCH, N, 8, CH), jnp.float32),
            jax.ShapeDtypeStruct((NCH, N, 8, CH), jnp.float32),
        ],
    )(tok8, pos_table, row_table)
    return tok8, negmu, rstd


def _splat(vec, j):
    idx = (jnp.zeros((NLANE, 1), jnp.int32) + j).astype(jnp.int32)
    dn = lax.GatherDimensionNumbers(
        offset_dims=(), collapsed_slice_dims=(0,), start_index_map=(0,))
    return lax.gather(vec, idx, dn, (1,),
                      mode=lax.GatherScatterMode.PROMISE_IN_BOUNDS)


def _make_sc_kernel(BN, N):
    NW = BN // RPW
    mesh = plsc.VectorSubcoreMesh(core_axis_name="c", subcore_axis_name="s")

    @functools.partial(
        pl.kernel,
        out_type=jax.ShapeDtypeStruct((BN, L, D), jnp.float32),
        mesh=mesh,
        scratch_types=[
            pltpu.VMEM((CH, D), jnp.float32),
            pltpu.VMEM((2, CH, D), jnp.float32),
            pltpu.VMEM((5 * D,), jnp.float32),
            pltpu.VMEM((8, D), jnp.float32),
            pltpu.VMEM((RPW, D), jnp.float32),
            pltpu.VMEM((RPW * 8 * CH,), jnp.float32),
            pltpu.VMEM((RPW * 8 * CH,), jnp.float32),
            pltpu.VMEM((RPW, CH), jnp.int32),
            pltpu.VMEM((D,), jnp.float32),
            pltpu.VMEM((D,), jnp.float32),
            pltpu.SemaphoreType.DMA,
            pltpu.SemaphoreType.DMA,
            pltpu.SemaphoreType.DMA,
        ],
        compiler_params=pltpu.CompilerParams(needs_layout_passes=False),
    )
    def sc_kernel(msa_hbm, pos_hbm, tok_hbm, row_hbm, gamma_hbm, beta_hbm,
                  negmu_hbm, rstd_hbm, out_hbm,
                  pos_v, out_v, rowtok_v, tok_v, rows_v, negmu_v, rstd_v,
                  msa_v, gamma_v, beta_v, semS, semA, semB):
        wid = lax.axis_index("s") * 2 + lax.axis_index("c")
        r0 = wid * RPW
        n0 = lax.rem(r0, N)

        pltpu.sync_copy(tok_hbm, tok_v)
        pltpu.sync_copy(row_hbm.at[pl.ds(n0, RPW)], rows_v)
        pltpu.sync_copy(gamma_hbm, gamma_v)
        pltpu.sync_copy(beta_hbm, beta_v)

        gk = [gamma_v[pl.ds(k * NLANE, NLANE)] for k in range(D // NLANE)]
        bk = [beta_v[pl.ds(k * NLANE, NLANE)] for k in range(D // NLANE)]
        iota = lax.iota(jnp.int32, NLANE)
        zeros16 = jnp.zeros((NLANE,), jnp.int32)

        def stage_chunk(c):
            h1 = pltpu.async_copy(pos_hbm.at[pl.ds(c * CH, CH)], pos_v, semS)
            h2 = pltpu.async_copy(msa_hbm.at[c, pl.ds(r0, RPW)], msa_v, semS)
            h3 = pltpu.async_copy(
                negmu_hbm.at[c, pl.ds(n0 * (8 * CH), RPW * 8 * CH)], negmu_v, semS)
            h4 = pltpu.async_copy(
                rstd_hbm.at[c, pl.ds(n0 * (8 * CH), RPW * 8 * CH)], rstd_v, semS)
            h1.wait(); h2.wait(); h3.wait(); h4.wait()

        def build_rowtok(lr):
            def tbody(t, carry):
                for k in range(D // NLANE):
                    sl = pl.ds(k * NLANE, NLANE)
                    rowtok_v[pl.ds(t * D + k * NLANE, NLANE)] = (
                        tok_v[t, sl] + rows_v[lr, sl])
                return carry
            lax.fori_loop(0, 5, tbody, 0, unroll=False)

        def compute_unit(lr, s):
            build_rowtok(lr)
            lrs = zeros16 + lr

            @plsc.parallel_loop(0, CH // NLANE, unroll=2)
            def gbody(g):
                l0 = g * NLANE
                tvec = msa_v[lr, pl.ds(l0, NLANE)]
                lvec = iota + l0
                sidx = (lrs * 8 + tvec) * CH + lvec
                nmv = plsc.load_gather(negmu_v, [sidx])
                rsv = plsc.load_gather(rstd_v, [sidx])

                @plsc.parallel_loop(0, NLANE, unroll=8)
                def jbody(j):
                    l = l0 + j
                    nm = _splat(nmv, j)
                    rs = _splat(rsv, j)
                    ts = _splat(tvec, j)
                    base = ts * D + iota
                    for k in range(D // NLANE):
                        vp = pos_v[l, pl.ds(k * NLANE, NLANE)]
                        vt = plsc.load_gather(rowtok_v, [base + (k * NLANE)])
                        v = (vp + vt + nm) * rs
                        v = v * gk[k] + bk[k]
                        out_v[s, l, pl.ds(k * NLANE, NLANE)] = v

        def fire(lr, c, s, sem):
            return pltpu.async_copy(
                out_v.at[s], out_hbm.at[r0 + lr, pl.ds(c * CH, CH)], sem)

        def drain(s, sem):
            pltpu.make_async_copy(
                out_v.at[s], out_hbm.at[r0, pl.ds(0, CH)], sem).wait()

        stage_chunk(0)
        compute_unit(0, 0)
        fire(0, 0, 0, semA)
        compute_unit(1, 1)
        fire(1, 0, 1, semB)

        def pbody(p, carry):
            c = lax.div(p, NCH)
            q = lax.rem(p, NCH)

            @pl.when(q == 0)
            def _():
                stage_chunk(c)

            lrA = 2 * q
            drain(0, semA)
            compute_unit(lrA, 0)
            fire(lrA, c, 0, semA)
            drain(1, semB)
            compute_unit(lrA + 1, 1)
            fire(lrA + 1, c, 1, semB)
            return carry

        lax.fori_loop(1, NCH * (RPW // 2), pbody, 0, unroll=False)
        drain(0, semA)
        drain(1, semB)

    return sc_kernel


@jax.jit
def _msa_embed(msa, token_table, pos_table, row_table, gamma, beta):
    B, N, Lx = msa.shape
    BN = B * N
    tok8, negmu, rstd = _stats_tables(token_table, pos_table, row_table)
    msa_c = (msa.astype(jnp.int32).reshape(BN, NCH, CH)
             .transpose(1, 0, 2))
    negmu = negmu.reshape(NCH, N * 8 * CH)
    rstd = rstd.reshape(NCH, N * 8 * CH)
    sc = _make_sc_kernel(BN, N)
    out = sc(msa_c, pos_table, tok8, row_table, gamma, beta, negmu, rstd)
    return out.reshape(B, N, Lx, D)


def kernel(msa, token_table, pos_table, row_table, gamma, beta):
    return _msa_embed(msa, token_table, pos_table, row_table, gamma, beta)


# speedup vs baseline: 1.1756x; 1.1756x over previous
import functools

import jax
import jax.numpy as jnp
from jax import lax
from jax.experimental import pallas as pl
from jax.experimental.pallas import tpu as pltpu
from jax.experimental.pallas import tpu_sc as plsc

EPS = 1e-5
D = 256
L = 512
CH = 128
NCH = L // CH
RPW = 8
NLANE = 16


def _stats_body(tok_ref, pos_ref, row_ref, negmu_ref, rstd_ref):
    tok = tok_ref[...]
    pos = pos_ref[...]
    row = row_ref[...]
    muT = jnp.mean(tok, axis=1)
    muP = jnp.mean(pos, axis=1)
    muR = jnp.mean(row, axis=1)
    St = jnp.sum(tok * tok, axis=1)
    Sp = jnp.sum(pos * pos, axis=1)
    Sr = jnp.sum(row * row, axis=1)
    cdims = (((1,), (1,)), ((), ()))
    PT = lax.dot_general(tok, pos, cdims, preferred_element_type=jnp.float32)
    PR = lax.dot_general(row, pos, cdims, preferred_element_type=jnp.float32)
    RT = lax.dot_general(row, tok, cdims, preferred_element_type=jnp.float32)
    mu = muR[:, None, None] + muT[None, :, None] + muP[None, None, :]
    S = (Sr[:, None, None] + St[None, :, None] + Sp[None, None, :]
         + 2.0 * (RT[:, :, None] + PT[None, :, :] + PR[:, None, :]))
    var = S * (1.0 / D) - mu * mu
    negmu_ref[0] = -mu
    rstd_ref[0] = lax.rsqrt(var + EPS)


def _stats_tables(token_table, pos_table, row_table):
    V = token_table.shape[0]
    N = row_table.shape[0]
    tok8 = jnp.zeros((8, D), jnp.float32).at[:V].set(token_table)
    negmu, rstd = pl.pallas_call(
        _stats_body,
        grid=(NCH,),
        in_specs=[
            pl.BlockSpec((8, D), lambda c: (0, 0)),
            pl.BlockSpec((CH, D), lambda c: (c, 0)),
            pl.BlockSpec((N, D), lambda c: (0, 0)),
        ],
        out_specs=[
            pl.BlockSpec((1, N, 8, CH), lambda c: (c, 0, 0, 0)),
            pl.BlockSpec((1, N, 8, CH), lambda c: (c, 0, 0, 0)),
        ],
        out_shape=[
            jax.ShapeDtypeStruct((NCH, N, 8, CH), jnp.float32),
            jax.ShapeDtypeStruct((NCH, N, 8, CH), jnp.float32),
        ],
    )(tok8, pos_table, row_table)
    return tok8, negmu, rstd


def _splat(vec, j):
    idx = (jnp.zeros((NLANE, 1), jnp.int32) + j).astype(jnp.int32)
    dn = lax.GatherDimensionNumbers(
        offset_dims=(), collapsed_slice_dims=(0,), start_index_map=(0,))
    return lax.gather(vec, idx, dn, (1,),
                      mode=lax.GatherScatterMode.PROMISE_IN_BOUNDS)


def _make_sc_kernel(BN, N):
    NW = BN // RPW
    mesh = plsc.VectorSubcoreMesh(core_axis_name="c", subcore_axis_name="s")

    @functools.partial(
        pl.kernel,
        out_type=jax.ShapeDtypeStruct((BN, L, D), jnp.float32),
        mesh=mesh,
        scratch_types=[
            pltpu.VMEM((CH, D), jnp.float32),
            pltpu.VMEM((2, CH, D), jnp.float32),
            pltpu.VMEM((5 * D,), jnp.float32),
            pltpu.VMEM((8, D), jnp.float32),
            pltpu.VMEM((RPW, D), jnp.float32),
            pltpu.VMEM((RPW * 8 * CH,), jnp.float32),
            pltpu.VMEM((RPW * 8 * CH,), jnp.float32),
            pltpu.VMEM((RPW, CH), jnp.int32),
            pltpu.VMEM((D,), jnp.float32),
            pltpu.VMEM((D,), jnp.float32),
            pltpu.SemaphoreType.DMA,
            pltpu.SemaphoreType.DMA,
            pltpu.SemaphoreType.DMA,
        ],
        compiler_params=pltpu.CompilerParams(needs_layout_passes=False),
    )
    def sc_kernel(msa_hbm, pos_hbm, tok_hbm, row_hbm, gamma_hbm, beta_hbm,
                  negmu_hbm, rstd_hbm, out_hbm,
                  pos_v, out_v, rowtok_v, tok_v, rows_v, negmu_v, rstd_v,
                  msa_v, gamma_v, beta_v, semS, semA, semB):
        wid = lax.axis_index("s") * 2 + lax.axis_index("c")
        r0 = wid * RPW
        n0 = lax.rem(r0, N)

        pltpu.sync_copy(tok_hbm, tok_v)
        pltpu.sync_copy(row_hbm.at[pl.ds(n0, RPW)], rows_v)
        pltpu.sync_copy(gamma_hbm, gamma_v)
        pltpu.sync_copy(beta_hbm, beta_v)

        gk = [gamma_v[pl.ds(k * NLANE, NLANE)] for k in range(D // NLANE)]
        bk = [beta_v[pl.ds(k * NLANE, NLANE)] for k in range(D // NLANE)]
        iota = lax.iota(jnp.int32, NLANE)
        zeros16 = jnp.zeros((NLANE,), jnp.int32)

        def stage_chunk(c):
            h1 = pltpu.async_copy(pos_hbm.at[pl.ds(c * CH, CH)], pos_v, semS)
            h2 = pltpu.async_copy(msa_hbm.at[c, pl.ds(r0, RPW)], msa_v, semS)
            h3 = pltpu.async_copy(
                negmu_hbm.at[c, pl.ds(n0 * (8 * CH), RPW * 8 * CH)], negmu_v, semS)
            h4 = pltpu.async_copy(
                rstd_hbm.at[c, pl.ds(n0 * (8 * CH), RPW * 8 * CH)], rstd_v, semS)
            h1.wait(); h2.wait(); h3.wait(); h4.wait()

        def build_rowtok(lr):
            def tbody(t, carry):
                for k in range(D // NLANE):
                    sl = pl.ds(k * NLANE, NLANE)
                    rowtok_v[pl.ds(t * D + k * NLANE, NLANE)] = (
                        tok_v[t, sl] + rows_v[lr, sl])
                return carry
            lax.fori_loop(0, 5, tbody, 0, unroll=False)

        def compute_unit(lr, s):
            build_rowtok(lr)
            lrs = zeros16 + lr

            @plsc.parallel_loop(0, CH // NLANE)
            def gbody(g):
                l0 = g * NLANE
                tvec = msa_v[lr, pl.ds(l0, NLANE)]
                lvec = iota + l0
                sidx = (lrs * 8 + tvec) * CH + lvec
                nmv = plsc.load_gather(negmu_v, [sidx])
                rsv = plsc.load_gather(rstd_v, [sidx])

                @plsc.parallel_loop(0, NLANE, unroll=8)
                def jbody(j):
                    l = l0 + j
                    nm = _splat(nmv, j)
                    rs = _splat(rsv, j)
                    ts = _splat(tvec, j)
                    base = ts * D + iota
                    for k in range(D // NLANE):
                        vp = pos_v[l, pl.ds(k * NLANE, NLANE)]
                        vt = plsc.load_gather(rowtok_v, [base + (k * NLANE)])
                        v = (vp + vt + nm) * rs
                        v = v * gk[k] + bk[k]
                        out_v[s, l, pl.ds(k * NLANE, NLANE)] = v

        def fire(lr, c, s, sem):
            return pltpu.async_copy(
                out_v.at[s], out_hbm.at[r0 + lr, pl.ds(c * CH, CH)], sem)

        def drain(s, sem):
            pltpu.make_async_copy(
                out_v.at[s], out_hbm.at[r0, pl.ds(0, CH)], sem).wait()

        stage_chunk(0)
        compute_unit(0, 0)
        fire(0, 0, 0, semA)
        compute_unit(1, 1)
        fire(1, 0, 1, semB)

        def pbody(p, carry):
            c = lax.div(p, NCH)
            q = lax.rem(p, NCH)

            @pl.when(q == 0)
            def _():
                stage_chunk(c)

            lrA = 2 * q
            drain(0, semA)
            compute_unit(lrA, 0)
            fire(lrA, c, 0, semA)
            drain(1, semB)
            compute_unit(lrA + 1, 1)
            fire(lrA + 1, c, 1, semB)
            return carry

        lax.fori_loop(1, NCH * (RPW // 2), pbody, 0, unroll=False)
        drain(0, semA)
        drain(1, semB)

    return sc_kernel


@jax.jit
def _msa_embed(msa, token_table, pos_table, row_table, gamma, beta):
    B, N, Lx = msa.shape
    BN = B * N
    tok8, negmu, rstd = _stats_tables(token_table, pos_table, row_table)
    msa_c = (msa.astype(jnp.int32).reshape(BN, NCH, CH)
             .transpose(1, 0, 2))
    negmu = negmu.reshape(NCH, N * 8 * CH)
    rstd = rstd.reshape(NCH, N * 8 * CH)
    sc = _make_sc_kernel(BN, N)
    out = sc(msa_c, pos_table, tok8, row_table, gamma, beta, negmu, rstd)
    return out.reshape(B, N, Lx, D)


def kernel(msa, token_table, pos_table, row_table, gamma, beta):
    return _msa_embed(msa, token_table, pos_table, row_table, gamma, beta)
